# initial kernel scaffold (unmeasured)
import functools

import jax
import jax.numpy as jnp
from jax import lax
from jax.experimental import pallas as pl
from jax.experimental.pallas import tpu as pltpu

BM = 512
EPS = 1e-5


def kernel(x, dy, gamma):
    m_per, d = x.shape
    n_steps = m_per // BM

    def body(x_ref, dy_ref, gamma_ref, out_ref, comm_ref, send_sem, recv_sem):
        step = pl.program_id(0)

        @pl.when(step == 0)
        def _():
            out_ref[...] = jnp.zeros_like(out_ref)

        xb = x_ref[...].astype(jnp.float32)
        dyb = dy_ref[...].astype(jnp.float32)
        mu = jnp.mean(xb, axis=1, keepdims=True)
        var = jnp.mean(xb * xb, axis=1, keepdims=True) - mu * mu
        rstd = lax.rsqrt(var + EPS)
        xhat = (xb - mu) * rstd
        dgamma = jnp.sum(dyb * xhat, axis=0)
        dbeta = jnp.sum(dyb, axis=0)
        out_ref[...] += jnp.stack([dgamma, dbeta])

        @pl.when(step == n_steps - 1)
        def _():
            my_x = lax.axis_index("x")
            my_y = lax.axis_index("y")
            my_z = lax.axis_index("z")
            peer = (my_x, 1 - my_y, my_z)

            barrier_sem = pltpu.get_barrier_semaphore()
            pl.semaphore_signal(
                barrier_sem, inc=1,
                device_id=peer, device_id_type=pl.DeviceIdType.MESH,
            )
            pl.semaphore_wait(barrier_sem, 1)

            comm_ref[0] = out_ref[...]
            rdma = pltpu.make_async_remote_copy(
                src_ref=comm_ref.at[0],
                dst_ref=comm_ref.at[1],
                send_sem=send_sem,
                recv_sem=recv_sem,
                device_id=peer,
                device_id_type=pl.DeviceIdType.MESH,
            )
            rdma.start()
            rdma.wait()
            out_ref[...] += comm_ref[1]

            @functools.partial(
                pl.run_scoped, sem2=pltpu.SemaphoreType.REGULAR
            )
            def _(sem2):
                pl.semaphore_signal(
                    sem2, inc=1,
                    device_id=peer, device_id_type=pl.DeviceIdType.MESH,
                )
                pl.semaphore_wait(sem2, 1)

    return pl.pallas_call(
        body,
        grid=(n_steps,),
        out_shape=jax.ShapeDtypeStruct((2, d), jnp.float32),
        in_specs=[
            pl.BlockSpec((BM, d), lambda i: (i, 0)),
            pl.BlockSpec((BM, d), lambda i: (i, 0)),
            pl.BlockSpec(memory_space=pltpu.ANY),
        ],
        out_specs=pl.BlockSpec((2, d), lambda i: (0, 0)),
        scratch_shapes=[
            pltpu.VMEM((2, 2, d), jnp.float32),
            pltpu.SemaphoreType.DMA,
            pltpu.SemaphoreType.DMA,
        ],
        compiler_params=pltpu.CompilerParams(collective_id=0),
    )(x, dy, gamma)


# baseline (device time: 29851 ns/iter reference)
import functools

import jax
import jax.numpy as jnp
from jax import lax
from jax.experimental import pallas as pl
from jax.experimental.pallas import tpu as pltpu

BM = 512
EPS = 1e-5


def kernel(x, dy, gamma):
    m_per, d = x.shape
    n_steps = m_per // BM

    def body(x_ref, dy_ref, gamma_ref, out_ref, comm_ref, send_sem, recv_sem):
        step = pl.program_id(0)

        @pl.when(step == 0)
        def _():
            out_ref[...] = jnp.zeros_like(out_ref)

        xb = x_ref[...].astype(jnp.float32)
        dyb = dy_ref[...].astype(jnp.float32)
        mu = jnp.mean(xb, axis=1, keepdims=True)
        var = jnp.mean(xb * xb, axis=1, keepdims=True) - mu * mu
        rstd = lax.rsqrt(var + EPS)
        xhat = (xb - mu) * rstd
        dgamma = jnp.sum(dyb * xhat, axis=0)
        dbeta = jnp.sum(dyb, axis=0)
        out_ref[...] += jnp.stack([dgamma, dbeta])

        @pl.when(step == n_steps - 1)
        def _():
            my_x = lax.axis_index("x")
            my_y = lax.axis_index("y")
            my_z = lax.axis_index("z")
            peer = (my_x, 1 - my_y, my_z)

            barrier_sem = pltpu.get_barrier_semaphore()
            pl.semaphore_signal(
                barrier_sem, inc=1,
                device_id=peer, device_id_type=pl.DeviceIdType.MESH,
            )
            pl.semaphore_wait(barrier_sem, 1)

            comm_ref[0] = out_ref[...]
            rdma = pltpu.make_async_remote_copy(
                src_ref=comm_ref.at[0],
                dst_ref=comm_ref.at[1],
                send_sem=send_sem,
                recv_sem=recv_sem,
                device_id=peer,
                device_id_type=pl.DeviceIdType.MESH,
            )
            rdma.start()
            rdma.wait()
            out_ref[...] += comm_ref[1]

            @functools.partial(
                pl.run_scoped, sem2=pltpu.SemaphoreType.REGULAR
            )
            def _(sem2):
                pl.semaphore_signal(
                    sem2, inc=1,
                    device_id=peer, device_id_type=pl.DeviceIdType.MESH,
                )
                pl.semaphore_wait(sem2, 1)

    return pl.pallas_call(
        body,
        grid=(n_steps,),
        out_shape=jax.ShapeDtypeStruct((2, d), jnp.float32),
        in_specs=[
            pl.BlockSpec((BM, d), lambda i: (i, 0)),
            pl.BlockSpec((BM, d), lambda i: (i, 0)),
            pl.BlockSpec(memory_space=pl.ANY),
        ],
        out_specs=pl.BlockSpec((2, d), lambda i: (0, 0)),
        scratch_shapes=[
            pltpu.VMEM((2, 2, d), jnp.float32),
            pltpu.SemaphoreType.DMA,
            pltpu.SemaphoreType.DMA,
        ],
        compiler_params=pltpu.CompilerParams(collective_id=0),
    )(x, dy, gamma)


# device time: 29108 ns/iter; 1.0255x vs baseline; 1.0255x over previous
import functools

import jax
import jax.numpy as jnp
from jax import lax
from jax.experimental import pallas as pl
from jax.experimental.pallas import tpu as pltpu

BM = 512
EPS = 1e-5


def kernel(x, dy, gamma):
    m_per, d = x.shape
    n_steps = m_per // BM

    def body(x_ref, dy_ref, gamma_ref, out_ref, comm_ref, send_sem, recv_sem):
        step = pl.program_id(0)

        @pl.when(step == 0)
        def _():
            out_ref[...] = jnp.zeros_like(out_ref)

        xb = x_ref[...].astype(jnp.float32)
        dyb = dy_ref[...].astype(jnp.float32)
        mu = jnp.mean(xb, axis=1, keepdims=True)
        var = jnp.mean(xb * xb, axis=1, keepdims=True) - mu * mu
        rstd = lax.rsqrt(var + EPS)
        xhat = (xb - mu) * rstd
        ones_row = jnp.ones((1, BM), jnp.float32)
        dgamma = jnp.dot(
            ones_row, dyb * xhat, preferred_element_type=jnp.float32
        )
        dbeta = jnp.dot(ones_row, dyb, preferred_element_type=jnp.float32)
        out_ref[...] += jnp.concatenate([dgamma, dbeta], axis=0)

        @pl.when(step == n_steps - 1)
        def _():
            my_x = lax.axis_index("x")
            my_y = lax.axis_index("y")
            my_z = lax.axis_index("z")
            peer = (my_x, 1 - my_y, my_z)

            barrier_sem = pltpu.get_barrier_semaphore()
            pl.semaphore_signal(
                barrier_sem, inc=1,
                device_id=peer, device_id_type=pl.DeviceIdType.MESH,
            )
            pl.semaphore_wait(barrier_sem, 1)

            comm_ref[0] = out_ref[...]
            rdma = pltpu.make_async_remote_copy(
                src_ref=comm_ref.at[0],
                dst_ref=comm_ref.at[1],
                send_sem=send_sem,
                recv_sem=recv_sem,
                device_id=peer,
                device_id_type=pl.DeviceIdType.MESH,
            )
            rdma.start()
            rdma.wait()
            out_ref[...] += comm_ref[1]

            @functools.partial(
                pl.run_scoped, sem2=pltpu.SemaphoreType.REGULAR
            )
            def _(sem2):
                pl.semaphore_signal(
                    sem2, inc=1,
                    device_id=peer, device_id_type=pl.DeviceIdType.MESH,
                )
                pl.semaphore_wait(sem2, 1)

    return pl.pallas_call(
        body,
        grid=(n_steps,),
        out_shape=jax.ShapeDtypeStruct((2, d), jnp.float32),
        in_specs=[
            pl.BlockSpec((BM, d), lambda i: (i, 0)),
            pl.BlockSpec((BM, d), lambda i: (i, 0)),
            pl.BlockSpec(memory_space=pl.ANY),
        ],
        out_specs=pl.BlockSpec((2, d), lambda i: (0, 0)),
        scratch_shapes=[
            pltpu.VMEM((2, 2, d), jnp.float32),
            pltpu.SemaphoreType.DMA,
            pltpu.SemaphoreType.DMA,
        ],
        compiler_params=pltpu.CompilerParams(collective_id=0),
    )(x, dy, gamma)


# device time: 28892 ns/iter; 1.0332x vs baseline; 1.0075x over previous
import functools

import jax
import jax.numpy as jnp
from jax import lax
from jax.experimental import pallas as pl
from jax.experimental.pallas import tpu as pltpu

BM = 512
EPS = 1e-5


def kernel(x, dy, gamma):
    m_per, d = x.shape
    n_steps = m_per // BM

    def body(x_ref, dy_ref, gamma_ref, out_ref, comm_ref, send_sem, recv_sem):
        step = pl.program_id(0)

        @pl.when(step == 0)
        def _():
            out_ref[...] = jnp.zeros_like(out_ref)

        xb = x_ref[...].astype(jnp.float32)
        dyb = dy_ref[...].astype(jnp.float32)
        mu = jnp.mean(xb, axis=1, keepdims=True)
        var = jnp.mean(xb * xb, axis=1, keepdims=True) - mu * mu
        rstd = lax.rsqrt(var + EPS)
        w_dy = jnp.concatenate(
            [-(rstd * mu), jnp.ones_like(rstd)], axis=1
        )
        g_p = jnp.dot(
            rstd.reshape(1, BM), xb * dyb,
            preferred_element_type=jnp.float32,
        )
        g_dy = jnp.dot(
            w_dy.T, dyb, preferred_element_type=jnp.float32
        )
        out_ref[0, :] += g_p[0, :] + g_dy[0, :]
        out_ref[1, :] += g_dy[1, :]

        @pl.when(step == n_steps - 1)
        def _():
            my_x = lax.axis_index("x")
            my_y = lax.axis_index("y")
            my_z = lax.axis_index("z")
            peer = (my_x, 1 - my_y, my_z)

            barrier_sem = pltpu.get_barrier_semaphore()
            pl.semaphore_signal(
                barrier_sem, inc=1,
                device_id=peer, device_id_type=pl.DeviceIdType.MESH,
            )
            pl.semaphore_wait(barrier_sem, 1)

            comm_ref[0] = out_ref[...]
            rdma = pltpu.make_async_remote_copy(
                src_ref=comm_ref.at[0],
                dst_ref=comm_ref.at[1],
                send_sem=send_sem,
                recv_sem=recv_sem,
                device_id=peer,
                device_id_type=pl.DeviceIdType.MESH,
            )
            rdma.start()
            rdma.wait()
            out_ref[...] += comm_ref[1]

            @functools.partial(
                pl.run_scoped, sem2=pltpu.SemaphoreType.REGULAR
            )
            def _(sem2):
                pl.semaphore_signal(
                    sem2, inc=1,
                    device_id=peer, device_id_type=pl.DeviceIdType.MESH,
                )
                pl.semaphore_wait(sem2, 1)

    return pl.pallas_call(
        body,
        grid=(n_steps,),
        out_shape=jax.ShapeDtypeStruct((2, d), jnp.float32),
        in_specs=[
            pl.BlockSpec((BM, d), lambda i: (i, 0)),
            pl.BlockSpec((BM, d), lambda i: (i, 0)),
            pl.BlockSpec(memory_space=pl.ANY),
        ],
        out_specs=pl.BlockSpec((2, d), lambda i: (0, 0)),
        scratch_shapes=[
            pltpu.VMEM((2, 2, d), jnp.float32),
            pltpu.SemaphoreType.DMA,
            pltpu.SemaphoreType.DMA,
        ],
        compiler_params=pltpu.CompilerParams(collective_id=0),
    )(x, dy, gamma)
